# Initial kernel scaffold; baseline (speedup 1.0000x reference)
import jax
import jax.numpy as jnp
from jax.experimental import pallas as pl


def kernel(inputs, codebook):
    raise NotImplementedError("write your pallas kernel here")


# speedup vs baseline: 2.1149x; 2.1149x over previous
import jax
import jax.numpy as jnp
from jax import lax
from jax.experimental import pallas as pl
from jax.experimental.pallas import tpu as pltpu
from jax.experimental.pallas import tpu_sc as plsc

_K = 1024
_D = 64
_N = 16384
_COMMIT = 0.25

_BT = 512
_NB = _N // _BT

_NC, _NS = 2, 16
_NW = _NC * _NS
_BW = _N // _NW
_CH = 128
_NCH = _BW // _CH


def _vq_tc_body(x_ref, c_ref, idx_ref, acc_ref):
    x = x_ref[...]
    c = c_ref[...]
    rn = jnp.sum(x * x, axis=1, keepdims=True)
    cn = jnp.sum(c * c, axis=1)
    mm = lax.dot_general(x, c, (((1,), (1,)), ((), ())),
                         preferred_element_type=jnp.float32)
    d = (rn + cn[None, :]) - 2.0 * mm
    dmin = jnp.min(d, axis=1, keepdims=True)
    kio = lax.broadcasted_iota(jnp.int32, (_BT, _K), 1)
    idx = jnp.min(jnp.where(d == dmin, kio, _K), axis=1)
    idx_ref[...] = idx[None, None, :]

    @pl.when(pl.program_id(0) == 0)
    def _init():
        acc_ref[...] = jnp.zeros_like(acc_ref)

    acc_ref[...] += jnp.sum(dmin).reshape(1, 1)


_vq_tc = pl.pallas_call(
    _vq_tc_body,
    grid=(_NB,),
    in_specs=[
        pl.BlockSpec((_BT, _D), lambda i: (i, 0)),
        pl.BlockSpec((_K, _D), lambda i: (0, 0)),
    ],
    out_specs=[
        pl.BlockSpec((1, 1, _BT), lambda i: (i, 0, 0)),
        pl.BlockSpec((1, 1), lambda i: (0, 0)),
    ],
    out_shape=[
        jax.ShapeDtypeStruct((_NB, 1, _BT), jnp.int32),
        jax.ShapeDtypeStruct((1, 1), jnp.float32),
    ],
    compiler_params=pltpu.CompilerParams(dimension_semantics=("arbitrary",)),
)


def _sc_gather_body(table_hbm, idx_hbm, out_hbm, idx_v, rows_v, sem):
    wid = lax.axis_index("s") * _NC + lax.axis_index("c")
    pltpu.sync_copy(idx_hbm.at[wid], idx_v)
    copies = [
        pltpu.async_copy(table_hbm.at[idx_v.at[j]], rows_v.at[j], sem)
        for j in range(_NCH)
    ]
    for cp in copies:
        cp.wait()
    pltpu.sync_copy(rows_v, out_hbm.at[wid])


import functools


@functools.lru_cache(maxsize=1)
def _sc_gather():
    return pl.kernel(
        _sc_gather_body,
        mesh=plsc.VectorSubcoreMesh(core_axis_name="c", subcore_axis_name="s"),
        out_type=jax.ShapeDtypeStruct((_NW, _NCH, _CH, 2 * _D), jnp.float32),
        scratch_types=[
            pltpu.VMEM((_NCH, _CH), jnp.int32),
            pltpu.VMEM((_NCH, _CH, 2 * _D), jnp.float32),
            pltpu.SemaphoreType.DMA,
        ],
    )


def kernel(inputs, codebook):
    idx3, acc = _vq_tc(inputs, codebook)
    idx = idx3.reshape(_N)
    cb_pad = jnp.pad(codebook, ((0, 0), (0, _D)))
    q4 = _sc_gather()(cb_pad, idx.reshape(_NW, _NCH, _CH))
    quantized = q4.reshape(_N, 2 * _D)[:, :_D]
    mean_d = acc[0, 0] / (_N * _D)
    loss = mean_d + _COMMIT * mean_d
    return quantized, loss, idx


# speedup vs baseline: 2.6837x; 1.2690x over previous
import jax
import jax.numpy as jnp
from jax import lax
from jax.experimental import pallas as pl
from jax.experimental.pallas import tpu as pltpu
from jax.experimental.pallas import tpu_sc as plsc

_K = 1024
_D = 64
_N = 16384
_COMMIT = 0.25

_BT = 512
_NB = _N // _BT

_NC, _NS = 2, 16
_NW = _NC * _NS
_BW = _N // _NW
_CH = 128
_NCH = _BW // _CH


def _vq_tc_body(x_ref, c_ref, idx_ref, acc_ref, cn_ref):
    x = x_ref[...]
    c = c_ref[...]
    rn = jnp.sum(x * x, axis=1, keepdims=True)

    @pl.when(pl.program_id(0) == 0)
    def _cn_once():
        cn_ref[...] = jnp.sum(c * c, axis=1)[None, :]

    cn = cn_ref[0, :]
    mm = lax.dot_general(x, c, (((1,), (1,)), ((), ())),
                         preferred_element_type=jnp.float32)
    d = (rn + cn[None, :]) - 2.0 * mm
    bestv = lax.slice(d, (0, 0), (_BT, 128))
    bestc = jnp.zeros((_BT, 128), jnp.float32)
    for ci in range(1, _K // 128):
        dc = lax.slice(d, (0, ci * 128), (_BT, (ci + 1) * 128))
        m = dc < bestv
        bestv = jnp.minimum(dc, bestv)
        bestc = jnp.where(m, float(ci), bestc)
    lane = lax.broadcasted_iota(jnp.int32, (_BT, 128), 1).astype(jnp.float32)
    cand = bestc * 128.0 + lane
    bvt = bestv.T
    cdt = cand.T
    gmin_t = jnp.min(bvt, axis=0, keepdims=True)
    idx_f = jnp.min(jnp.where(bvt == gmin_t, cdt, float(_K)), axis=0)
    idx_ref[...] = idx_f.astype(jnp.int32)[None, None, :]
    gmin = gmin_t

    @pl.when(pl.program_id(0) == 0)
    def _init():
        acc_ref[...] = jnp.zeros_like(acc_ref)

    acc_ref[...] += jnp.sum(gmin).reshape(1, 1)


_vq_tc = pl.pallas_call(
    _vq_tc_body,
    grid=(_NB,),
    in_specs=[
        pl.BlockSpec((_BT, _D), lambda i: (i, 0)),
        pl.BlockSpec((_K, _D), lambda i: (0, 0)),
    ],
    out_specs=[
        pl.BlockSpec((1, 1, _BT), lambda i: (i, 0, 0)),
        pl.BlockSpec((1, 1), lambda i: (0, 0)),
    ],
    out_shape=[
        jax.ShapeDtypeStruct((_NB, 1, _BT), jnp.int32),
        jax.ShapeDtypeStruct((1, 1), jnp.float32),
    ],
    scratch_shapes=[pltpu.VMEM((1, _K), jnp.float32)],
    compiler_params=pltpu.CompilerParams(dimension_semantics=("arbitrary",)),
)


def _sc_gather_body(table_hbm, idx_hbm, out_hbm, idx_v, rows_v, sem):
    wid = lax.axis_index("s") * _NC + lax.axis_index("c")
    pltpu.sync_copy(idx_hbm.at[wid], idx_v)
    copies = [
        pltpu.async_copy(table_hbm.at[idx_v.at[j]], rows_v.at[j], sem)
        for j in range(_NCH)
    ]
    for cp in copies:
        cp.wait()
    pltpu.sync_copy(rows_v, out_hbm.at[wid])


import functools


@functools.lru_cache(maxsize=1)
def _sc_gather():
    return pl.kernel(
        _sc_gather_body,
        mesh=plsc.VectorSubcoreMesh(core_axis_name="c", subcore_axis_name="s"),
        out_type=jax.ShapeDtypeStruct((_NW, _NCH, _CH, 2 * _D), jnp.float32),
        scratch_types=[
            pltpu.VMEM((_NCH, _CH), jnp.int32),
            pltpu.VMEM((_NCH, _CH, 2 * _D), jnp.float32),
            pltpu.SemaphoreType.DMA,
        ],
    )


def kernel(inputs, codebook):
    idx3, acc = _vq_tc(inputs, codebook)
    idx = idx3.reshape(_N)
    cb_pad = jnp.pad(codebook, ((0, 0), (0, _D)))
    q4 = _sc_gather()(cb_pad, idx.reshape(_NW, _NCH, _CH))
    quantized = q4.reshape(_N, 2 * _D)[:, :_D]
    mean_d = acc[0, 0] / (_N * _D)
    loss = mean_d + _COMMIT * mean_d
    return quantized, loss, idx
